# baseline (device time: 224982 ns/iter reference)
import jax
import jax.numpy as jnp
from jax import lax
from jax.experimental import pallas as pl
from jax.experimental.pallas import tpu as pltpu

N_DEV = 4
B = 2
S_LOC = 512
S = 2048
D = 1024
H_LOC = 8
DH = 128
S_BLK = 512
SCALE = 0.08838834764831843


def _neighbor_barrier(left, right):
    barrier = pltpu.get_barrier_semaphore()
    for nbr in (left, right):
        pl.semaphore_signal(
            barrier, inc=1, device_id=(nbr,),
            device_id_type=pl.DeviceIdType.MESH,
        )
    pl.semaphore_wait(barrier, 2)


def _rope_piece(t, offset):
    R = t.shape[0]
    pos = (lax.broadcasted_iota(jnp.int32, (R, DH), 0) + offset).astype(
        jnp.float32
    )
    d = lax.broadcasted_iota(jnp.int32, (R, DH), 1)
    half = (d // 2).astype(jnp.float32)
    inv = jnp.exp(-jnp.log(10000.0) * (half * (2.0 / DH)))
    ang = pos * inv
    l = jnp.concatenate([t[:, 1:], t[:, :1]], axis=1)
    r = jnp.concatenate([t[:, -1:], t[:, :-1]], axis=1)
    tr = jnp.where(d % 2 == 0, -l, r)
    return t * jnp.cos(ang) + tr * jnp.sin(ang)


def _fused_body(x_ref, wq_ref, wk_ref, wv_ref, wo_ref, out_ref,
                xg, kbuf, vbuf, qgbuf, rsbuf, sendbuf, ctxbuf,
                ag_send, ag_recv, rs_send, rs_recv):
    f32 = jnp.float32
    bf16 = jnp.bfloat16
    my = lax.axis_index("i")
    left = lax.rem(my + N_DEV - 1, N_DEV)
    right = lax.rem(my + 1, N_DEV)

    _neighbor_barrier(left, right)

    def kv_for_chunk(c):
        rows = pl.ds(c * S_LOC, S_LOC)
        for b in range(B):
            xcb = xg[c, b]
            kf = jnp.dot(xcb, wk_ref[...], preferred_element_type=f32)
            for hh in range(H_LOC):
                piece = _rope_piece(kf[:, hh * DH:(hh + 1) * DH], c * S_LOC)
                kbuf[b, rows, hh * DH:(hh + 1) * DH] = piece.astype(bf16)
            vf = jnp.dot(xcb, wv_ref[...], preferred_element_type=f32)
            vbuf[b, rows, :] = vf.astype(bf16)
            qf = jnp.dot(xcb, wq_ref[...], preferred_element_type=f32)
            for hh in range(H_LOC):
                piece = _rope_piece(qf[:, hh * DH:(hh + 1) * DH], c * S_LOC)
                qgbuf[b, rows, hh * DH:(hh + 1) * DH] = (
                    piece * SCALE
                ).astype(bf16)

    xg[my] = x_ref[...]
    for h in range(N_DEV - 1):
        origin = lax.rem(my + N_DEV - h, N_DEV)
        rdma = pltpu.make_async_remote_copy(
            src_ref=xg.at[origin],
            dst_ref=xg.at[origin],
            send_sem=ag_send.at[h],
            recv_sem=ag_recv.at[h],
            device_id=(right,),
            device_id_type=pl.DeviceIdType.MESH,
        )
        rdma.start()
        kv_for_chunk(origin)
        rdma.wait()
    kv_for_chunk(lax.rem(my + 1, N_DEV))

    def chunk_contrib_b(c, b, write_out):
        for sb in range(S_LOC // S_BLK):
            rows = pl.ds(sb * S_BLK, S_BLK)

            def hbody(h, _):
                hs = pl.ds(h * DH, DH)
                k_bh = kbuf[b, :, hs]
                v_bh = vbuf[b, :, hs]
                qs = qgbuf[b, pl.ds(c * S_LOC + sb * S_BLK, S_BLK), hs]
                s = lax.dot_general(
                    qs, k_bh, (((1,), (1,)), ((), ())),
                    preferred_element_type=f32,
                )
                p = jnp.exp(s.astype(bf16))
                denom = jnp.sum(
                    p, axis=1, keepdims=True, dtype=f32
                )
                ctx_un = jnp.dot(p, v_bh, preferred_element_type=f32)
                ctxbuf[:, hs] = (ctx_un * (1.0 / denom)).astype(bf16)
                return 0

            lax.fori_loop(0, H_LOC, hbody, 0)
            contrib = jnp.dot(
                ctxbuf[...], wo_ref[...], preferred_element_type=f32
            )
            if write_out:
                out_ref[b, rows, :] = contrib
            else:
                sendbuf[b, rows, :] = contrib.astype(bf16)

    def rs_rdma(idx):
        return pltpu.make_async_remote_copy(
            src_ref=sendbuf,
            dst_ref=rsbuf.at[idx],
            send_sem=rs_send.at[idx],
            recv_sem=rs_recv.at[idx],
            device_id=(right,),
            device_id_type=pl.DeviceIdType.MESH,
        )

    c = lax.rem(my + N_DEV - 1, N_DEV)
    for b in range(B):
        chunk_contrib_b(c, b, False)
    rs0 = rs_rdma(0)
    rs0.start()

    c = lax.rem(my + N_DEV - 2, N_DEV)
    for b in range(B):
        chunk_contrib_b(c, b, False)
    rs0.wait()
    sendbuf[...] = (
        sendbuf[...].astype(f32) + rsbuf[0].astype(f32)
    ).astype(bf16)
    rs1 = rs_rdma(1)
    rs1.start()

    c = lax.rem(my + N_DEV - 3, N_DEV)
    for b in range(B):
        chunk_contrib_b(c, b, False)
    rs1.wait()
    sendbuf[...] = (
        sendbuf[...].astype(f32) + rsbuf[1].astype(f32)
    ).astype(bf16)
    rs2 = rs_rdma(2)
    rs2.start()

    for b in range(B):
        chunk_contrib_b(my, b, True)
    rs2.wait()
    out_ref[...] = out_ref[...] + rsbuf[2].astype(f32)


def kernel(x, Wq, Wk, Wv, Wo):
    bf16 = jnp.bfloat16
    return pl.pallas_call(
        _fused_body,
        out_shape=jax.ShapeDtypeStruct((B, S_LOC, D), jnp.float32),
        in_specs=[pl.BlockSpec(memory_space=pltpu.VMEM)] * 5,
        out_specs=pl.BlockSpec(memory_space=pltpu.VMEM),
        scratch_shapes=[
            pltpu.VMEM((N_DEV, B, S_LOC, D), bf16),
            pltpu.VMEM((B, S, D), bf16),
            pltpu.VMEM((B, S, D), bf16),
            pltpu.VMEM((B, S, D), bf16),
            pltpu.VMEM((N_DEV - 1, B, S_LOC, D), bf16),
            pltpu.VMEM((B, S_LOC, D), bf16),
            pltpu.VMEM((S_BLK, D), bf16),
            pltpu.SemaphoreType.DMA((N_DEV - 1,)),
            pltpu.SemaphoreType.DMA((N_DEV - 1,)),
            pltpu.SemaphoreType.DMA((N_DEV - 1,)),
            pltpu.SemaphoreType.DMA((N_DEV - 1,)),
        ],
        compiler_params=pltpu.CompilerParams(
            collective_id=0,
            vmem_limit_bytes=64 * 1024 * 1024,
        ),
    )(
        x.astype(bf16),
        Wq.astype(bf16),
        Wk.astype(bf16),
        Wv.astype(bf16),
        Wo.astype(bf16),
    )


# device time: 217372 ns/iter; 1.0350x vs baseline; 1.0350x over previous
import jax
import jax.numpy as jnp
from jax import lax
from jax.experimental import pallas as pl
from jax.experimental.pallas import tpu as pltpu

N_DEV = 4
B = 2
S_LOC = 512
S = 2048
D = 1024
H_LOC = 8
DH = 128
S_BLK = 512
SCALE = 0.08838834764831843


def _neighbor_barrier(left, right):
    barrier = pltpu.get_barrier_semaphore()
    for nbr in (left, right):
        pl.semaphore_signal(
            barrier, inc=1, device_id=(nbr,),
            device_id_type=pl.DeviceIdType.MESH,
        )
    pl.semaphore_wait(barrier, 2)


def _rope_piece(t, offset):
    R = t.shape[0]
    pos = (lax.broadcasted_iota(jnp.int32, (R, DH), 0) + offset).astype(
        jnp.float32
    )
    d = lax.broadcasted_iota(jnp.int32, (R, DH), 1)
    half = (d // 2).astype(jnp.float32)
    inv = jnp.exp(-jnp.log(10000.0) * (half * (2.0 / DH)))
    ang = pos * inv
    l = jnp.concatenate([t[:, 1:], t[:, :1]], axis=1)
    r = jnp.concatenate([t[:, -1:], t[:, :-1]], axis=1)
    tr = jnp.where(d % 2 == 0, -l, r)
    return t * jnp.cos(ang) + tr * jnp.sin(ang)


def _fused_body(x_ref, wq_ref, wk_ref, wv_ref, wo_ref, out_ref,
                xg, kbuf, vbuf, qgbuf, rsbuf, sendbuf, ctxbuf,
                ag_send, ag_recv, rs_send, rs_recv):
    f32 = jnp.float32
    bf16 = jnp.bfloat16
    my = lax.axis_index("i")
    left = lax.rem(my + N_DEV - 1, N_DEV)
    right = lax.rem(my + 1, N_DEV)

    _neighbor_barrier(left, right)

    def kv_for_chunk(c):
        rows = pl.ds(c * S_LOC, S_LOC)
        for b in range(B):
            xcb = xg[c, b]
            kf = jnp.dot(xcb, wk_ref[...], preferred_element_type=f32)
            for hh in range(H_LOC):
                piece = _rope_piece(kf[:, hh * DH:(hh + 1) * DH], c * S_LOC)
                kbuf[b, rows, hh * DH:(hh + 1) * DH] = piece.astype(bf16)
            vf = jnp.dot(xcb, wv_ref[...], preferred_element_type=f32)
            vbuf[b, rows, :] = vf.astype(bf16)
            qf = jnp.dot(xcb, wq_ref[...], preferred_element_type=f32)
            for hh in range(H_LOC):
                piece = _rope_piece(qf[:, hh * DH:(hh + 1) * DH], c * S_LOC)
                qgbuf[b, rows, hh * DH:(hh + 1) * DH] = (
                    piece * SCALE
                ).astype(bf16)

    xg[my] = x_ref[...]
    for h in range(N_DEV - 1):
        origin = lax.rem(my + N_DEV - h, N_DEV)
        rdma = pltpu.make_async_remote_copy(
            src_ref=xg.at[origin],
            dst_ref=xg.at[origin],
            send_sem=ag_send.at[h],
            recv_sem=ag_recv.at[h],
            device_id=(right,),
            device_id_type=pl.DeviceIdType.MESH,
        )
        rdma.start()
        kv_for_chunk(origin)
        rdma.wait()
    kv_for_chunk(lax.rem(my + 1, N_DEV))

    def chunk_contrib_b(c, b, write_out):
        for sb in range(S_LOC // S_BLK):
            rows = pl.ds(sb * S_BLK, S_BLK)

            def one_head(h):
                hs = pl.ds(h * DH, DH)
                k_bh = kbuf[b, :, hs]
                v_bh = vbuf[b, :, hs]
                qs = qgbuf[b, pl.ds(c * S_LOC + sb * S_BLK, S_BLK), hs]
                s = lax.dot_general(
                    qs, k_bh, (((1,), (1,)), ((), ())),
                    preferred_element_type=f32,
                )
                p = jnp.exp(s.astype(bf16))
                denom = jnp.sum(
                    p, axis=1, keepdims=True, dtype=f32
                )
                ctx_un = jnp.dot(p, v_bh, preferred_element_type=f32)
                ctxbuf[:, hs] = (ctx_un * (1.0 / denom)).astype(bf16)

            def hbody(hp, _):
                one_head(hp * 2)
                one_head(hp * 2 + 1)
                return 0

            lax.fori_loop(0, H_LOC // 2, hbody, 0)
            contrib = jnp.dot(
                ctxbuf[...], wo_ref[...], preferred_element_type=f32
            )
            if write_out:
                out_ref[b, rows, :] = contrib
            else:
                sendbuf[b, rows, :] = contrib.astype(bf16)

    def rs_rdma(idx):
        return pltpu.make_async_remote_copy(
            src_ref=sendbuf,
            dst_ref=rsbuf.at[idx],
            send_sem=rs_send.at[idx],
            recv_sem=rs_recv.at[idx],
            device_id=(right,),
            device_id_type=pl.DeviceIdType.MESH,
        )

    c = lax.rem(my + N_DEV - 1, N_DEV)
    for b in range(B):
        chunk_contrib_b(c, b, False)
    rs0 = rs_rdma(0)
    rs0.start()

    c = lax.rem(my + N_DEV - 2, N_DEV)
    for b in range(B):
        chunk_contrib_b(c, b, False)
    rs0.wait()
    sendbuf[...] = (
        sendbuf[...].astype(f32) + rsbuf[0].astype(f32)
    ).astype(bf16)
    rs1 = rs_rdma(1)
    rs1.start()

    c = lax.rem(my + N_DEV - 3, N_DEV)
    for b in range(B):
        chunk_contrib_b(c, b, False)
    rs1.wait()
    sendbuf[...] = (
        sendbuf[...].astype(f32) + rsbuf[1].astype(f32)
    ).astype(bf16)
    rs2 = rs_rdma(2)
    rs2.start()

    for b in range(B):
        chunk_contrib_b(my, b, True)
    rs2.wait()
    out_ref[...] = out_ref[...] + rsbuf[2].astype(f32)


def kernel(x, Wq, Wk, Wv, Wo):
    bf16 = jnp.bfloat16
    return pl.pallas_call(
        _fused_body,
        out_shape=jax.ShapeDtypeStruct((B, S_LOC, D), jnp.float32),
        in_specs=[pl.BlockSpec(memory_space=pltpu.VMEM)] * 5,
        out_specs=pl.BlockSpec(memory_space=pltpu.VMEM),
        scratch_shapes=[
            pltpu.VMEM((N_DEV, B, S_LOC, D), bf16),
            pltpu.VMEM((B, S, D), bf16),
            pltpu.VMEM((B, S, D), bf16),
            pltpu.VMEM((B, S, D), bf16),
            pltpu.VMEM((N_DEV - 1, B, S_LOC, D), bf16),
            pltpu.VMEM((B, S_LOC, D), bf16),
            pltpu.VMEM((S_BLK, D), bf16),
            pltpu.SemaphoreType.DMA((N_DEV - 1,)),
            pltpu.SemaphoreType.DMA((N_DEV - 1,)),
            pltpu.SemaphoreType.DMA((N_DEV - 1,)),
            pltpu.SemaphoreType.DMA((N_DEV - 1,)),
        ],
        compiler_params=pltpu.CompilerParams(
            collective_id=0,
            vmem_limit_bytes=64 * 1024 * 1024,
        ),
    )(
        x.astype(bf16),
        Wq.astype(bf16),
        Wk.astype(bf16),
        Wv.astype(bf16),
        Wo.astype(bf16),
    )


# device time: 211146 ns/iter; 1.0655x vs baseline; 1.0295x over previous
import jax
import jax.numpy as jnp
from jax import lax
from jax.experimental import pallas as pl
from jax.experimental.pallas import tpu as pltpu

N_DEV = 4
B = 2
S_LOC = 512
S = 2048
D = 1024
H_LOC = 8
DH = 128
S_BLK = 512
SCALE = 0.08838834764831843


def _neighbor_barrier(left, right):
    barrier = pltpu.get_barrier_semaphore()
    for nbr in (left, right):
        pl.semaphore_signal(
            barrier, inc=1, device_id=(nbr,),
            device_id_type=pl.DeviceIdType.MESH,
        )
    pl.semaphore_wait(barrier, 2)


def _rope_piece(t, offset):
    R = t.shape[0]
    pos = (lax.broadcasted_iota(jnp.int32, (R, DH), 0) + offset).astype(
        jnp.float32
    )
    d = lax.broadcasted_iota(jnp.int32, (R, DH), 1)
    half = (d // 2).astype(jnp.float32)
    inv = jnp.exp(-jnp.log(10000.0) * (half * (2.0 / DH)))
    ang = pos * inv
    l = jnp.concatenate([t[:, 1:], t[:, :1]], axis=1)
    r = jnp.concatenate([t[:, -1:], t[:, :-1]], axis=1)
    tr = jnp.where(d % 2 == 0, -l, r)
    return t * jnp.cos(ang) + tr * jnp.sin(ang)


def _fused_body(x_ref, wq_ref, wk_ref, wv_ref, wo_ref, out_ref,
                xg, kbuf, vbuf, qgbuf, rsbuf, sendbuf, ctxbuf,
                ag_send, ag_recv, rs_send, rs_recv):
    f32 = jnp.float32
    bf16 = jnp.bfloat16
    my = lax.axis_index("i")
    left = lax.rem(my + N_DEV - 1, N_DEV)
    right = lax.rem(my + 1, N_DEV)

    _neighbor_barrier(left, right)

    def kv_for_chunk(c):
        rows = pl.ds(c * S_LOC, S_LOC)
        for b in range(B):
            xcb = xg[c, b]
            kf = jnp.dot(xcb, wk_ref[...], preferred_element_type=f32)
            for hh in range(H_LOC):
                piece = _rope_piece(kf[:, hh * DH:(hh + 1) * DH], c * S_LOC)
                kbuf[b, rows, hh * DH:(hh + 1) * DH] = piece.astype(bf16)
            vf = jnp.dot(xcb, wv_ref[...], preferred_element_type=f32)
            vbuf[b, rows, :] = vf.astype(bf16)
            qf = jnp.dot(xcb, wq_ref[...], preferred_element_type=f32)
            for hh in range(H_LOC):
                piece = _rope_piece(qf[:, hh * DH:(hh + 1) * DH], c * S_LOC)
                qgbuf[b, rows, hh * DH:(hh + 1) * DH] = (
                    piece * SCALE
                ).astype(bf16)

    xg[my] = x_ref[...]
    for h in range(N_DEV - 1):
        origin = lax.rem(my + N_DEV - h, N_DEV)
        rdma = pltpu.make_async_remote_copy(
            src_ref=xg.at[origin],
            dst_ref=xg.at[origin],
            send_sem=ag_send.at[h],
            recv_sem=ag_recv.at[h],
            device_id=(right,),
            device_id_type=pl.DeviceIdType.MESH,
        )
        rdma.start()
        kv_for_chunk(origin)
        rdma.wait()
    kv_for_chunk(lax.rem(my + 1, N_DEV))

    def chunk_contrib_b(c, b, write_out):
        for sb in range(S_LOC // S_BLK):
            rows = pl.ds(sb * S_BLK, S_BLK)

            def one_head(h):
                hs = pl.ds(h * DH, DH)
                k_bh = kbuf[b, :, hs]
                v_bh = vbuf[b, :, hs]
                qs = qgbuf[b, pl.ds(c * S_LOC + sb * S_BLK, S_BLK), hs]
                s = lax.dot_general(
                    qs, k_bh, (((1,), (1,)), ((), ())),
                    preferred_element_type=f32,
                )
                p = jnp.exp(s.astype(bf16))
                denom = jnp.sum(
                    p, axis=1, keepdims=True, dtype=f32
                )
                ctx_un = jnp.dot(p, v_bh, preferred_element_type=f32)
                ctxbuf[:, hs] = (ctx_un * (1.0 / denom)).astype(bf16)

            def hbody(hp, _):
                for j in range(4):
                    one_head(hp * 4 + j)
                return 0

            lax.fori_loop(0, H_LOC // 4, hbody, 0)
            contrib = jnp.dot(
                ctxbuf[...], wo_ref[...], preferred_element_type=f32
            )
            if write_out:
                out_ref[b, rows, :] = contrib
            else:
                sendbuf[b, rows, :] = contrib.astype(bf16)

    def rs_rdma(idx):
        return pltpu.make_async_remote_copy(
            src_ref=sendbuf,
            dst_ref=rsbuf.at[idx],
            send_sem=rs_send.at[idx],
            recv_sem=rs_recv.at[idx],
            device_id=(right,),
            device_id_type=pl.DeviceIdType.MESH,
        )

    c = lax.rem(my + N_DEV - 1, N_DEV)
    for b in range(B):
        chunk_contrib_b(c, b, False)
    rs0 = rs_rdma(0)
    rs0.start()

    c = lax.rem(my + N_DEV - 2, N_DEV)
    for b in range(B):
        chunk_contrib_b(c, b, False)
    rs0.wait()
    sendbuf[...] = (
        sendbuf[...].astype(f32) + rsbuf[0].astype(f32)
    ).astype(bf16)
    rs1 = rs_rdma(1)
    rs1.start()

    c = lax.rem(my + N_DEV - 3, N_DEV)
    for b in range(B):
        chunk_contrib_b(c, b, False)
    rs1.wait()
    sendbuf[...] = (
        sendbuf[...].astype(f32) + rsbuf[1].astype(f32)
    ).astype(bf16)
    rs2 = rs_rdma(2)
    rs2.start()

    for b in range(B):
        chunk_contrib_b(my, b, True)
    rs2.wait()
    out_ref[...] = out_ref[...] + rsbuf[2].astype(f32)


def kernel(x, Wq, Wk, Wv, Wo):
    bf16 = jnp.bfloat16
    return pl.pallas_call(
        _fused_body,
        out_shape=jax.ShapeDtypeStruct((B, S_LOC, D), jnp.float32),
        in_specs=[pl.BlockSpec(memory_space=pltpu.VMEM)] * 5,
        out_specs=pl.BlockSpec(memory_space=pltpu.VMEM),
        scratch_shapes=[
            pltpu.VMEM((N_DEV, B, S_LOC, D), bf16),
            pltpu.VMEM((B, S, D), bf16),
            pltpu.VMEM((B, S, D), bf16),
            pltpu.VMEM((B, S, D), bf16),
            pltpu.VMEM((N_DEV - 1, B, S_LOC, D), bf16),
            pltpu.VMEM((B, S_LOC, D), bf16),
            pltpu.VMEM((S_BLK, D), bf16),
            pltpu.SemaphoreType.DMA((N_DEV - 1,)),
            pltpu.SemaphoreType.DMA((N_DEV - 1,)),
            pltpu.SemaphoreType.DMA((N_DEV - 1,)),
            pltpu.SemaphoreType.DMA((N_DEV - 1,)),
        ],
        compiler_params=pltpu.CompilerParams(
            collective_id=0,
            vmem_limit_bytes=64 * 1024 * 1024,
        ),
    )(
        x.astype(bf16),
        Wq.astype(bf16),
        Wk.astype(bf16),
        Wv.astype(bf16),
        Wo.astype(bf16),
    )
